# baseline (speedup 1.0000x reference)
import jax
import jax.numpy as jnp
from jax import lax
from jax.experimental import pallas as pl
from jax.experimental.pallas import tpu as pltpu

_N = 100000
_L = 100
_R = 1024
_C = 128
_NPAD = _R * _C
_LR = 10
_NB = 17


def _bitonic_sort(x, row, lane, li):
    def phase(p, x):
        kk = jnp.int32(1) << p
        up = (li & kk) == 0

        def lane_pass(jj, x):
            t = jnp.int32(1) << (p - _LR - 1 - jj)
            bit = (lane & t) != 0
            fwd = pltpu.roll(x, _C - t, axis=1)
            bwd = pltpu.roll(x, t, axis=1)
            part = jnp.where(bit, bwd, fwd)
            keep_min = bit != up
            return jnp.where(keep_min, jnp.minimum(x, part), jnp.maximum(x, part))

        x = lax.fori_loop(0, jnp.maximum(p - _LR, 0), lane_pass, x)

        def row_pass(jj, x):
            s = jnp.int32(1) << (jnp.minimum(p, _LR) - 1 - jj)
            bit = (row & s) != 0
            fwd = pltpu.roll(x, _R - s, axis=0)
            bwd = pltpu.roll(x, s, axis=0)
            part = jnp.where(bit, bwd, fwd)
            keep_min = bit != up
            return jnp.where(keep_min, jnp.minimum(x, part), jnp.maximum(x, part))

        return lax.fori_loop(0, jnp.minimum(p, _LR), row_pass, x)

    return lax.fori_loop(1, _NB + 1, phase, x)


def _body(xs_ref, xt_ref, p_ref, out_ref):
    j = pl.program_id(0)
    row = lax.broadcasted_iota(jnp.int32, (_R, _C), 0)
    lane = lax.broadcasted_iota(jnp.int32, (_R, _C), 1)
    li = lane * _R + row
    valid = li < _N

    pv = p_ref[0]
    p0 = pv[:, 0:1]
    p1 = pv[:, 1:2]
    p2 = pv[:, 2:3]
    inv = lax.rsqrt(p0 * p0 + p1 * p1 + p2 * p2)
    q0 = p0 * inv
    q1 = p1 * inv
    q2 = p2 * inv

    inf = jnp.float32(jnp.inf)
    A = xs_ref[0] * q0 + xs_ref[1] * q1 + xs_ref[2] * q2
    B = xt_ref[0] * q0 + xt_ref[1] * q1 + xt_ref[2] * q2
    A = jnp.where(valid, A, inf)
    B = jnp.where(valid, B, inf)

    A = _bitonic_sort(A, row, lane, li)
    B = _bitonic_sort(B, row, lane, li)

    Brev = B
    for s in (1, 2, 4, 8, 16, 32, 64, 128, 256, 512):
        bit = (row & s) != 0
        Brev = jnp.where(bit, pltpu.roll(Brev, s, axis=0),
                         pltpu.roll(Brev, _R - s, axis=0))
    for t in (1, 2, 4, 8, 16, 32, 64):
        bit = (lane & t) != 0
        Brev = jnp.where(bit, pltpu.roll(Brev, t, axis=1),
                         pltpu.roll(Brev, _C - t, axis=1))
    pad = _NPAD - _N
    kc, kr = pad // _R, pad % _R
    Xa = pltpu.roll(Brev, _C - kc, axis=1)
    Xb = pltpu.roll(Brev, _C - kc - 1, axis=1)
    Bd = jnp.where(row < _R - kr,
                   pltpu.roll(Xa, _R - kr, axis=0),
                   pltpu.roll(Xb, _R - kr, axis=0))

    zero = jnp.float32(0)
    am = jnp.where(valid, A, zero)
    bm = jnp.where(valid, B, zero)
    dm = jnp.where(valid, Bd, zero)
    a2 = am * am
    b2 = bm * bm
    d2 = dm * dm

    X = jnp.sum(am)
    X2 = jnp.sum(a2)
    X3 = jnp.sum(a2 * am)
    X4 = jnp.sum(a2 * a2)
    Y = jnp.sum(bm)
    Y2 = jnp.sum(b2)
    Y3 = jnp.sum(b2 * bm)
    Y4 = jnp.sum(b2 * b2)

    n = jnp.float32(_N)
    p4x = 2 * n * X4 - 8 * X3 * X + 6 * X2 * X2
    p4y = 2 * n * Y4 - 8 * Y3 * Y + 6 * Y2 * Y2
    inv_n2 = jnp.float32(1.0) / (n * n)

    xy1 = jnp.sum(am * bm)
    xxy1 = jnp.sum(a2 * bm)
    xyy1 = jnp.sum(am * b2)
    xxyy1 = jnp.sum(a2 * b2)
    C2_1 = 2 * X2 * Y2 + 2 * (n * xxyy1 - 2 * Y * xxy1 - 2 * X * xyy1 + 2 * xy1 * xy1)
    l1 = (p4x + p4y - 2 * C2_1) * inv_n2

    xy2 = jnp.sum(am * dm)
    xxy2 = jnp.sum(a2 * dm)
    xyy2 = jnp.sum(am * d2)
    xxyy2 = jnp.sum(a2 * d2)
    C2_2 = 2 * X2 * Y2 + 2 * (n * xxyy2 - 2 * Y * xxy2 - 2 * X * xyy2 + 2 * xy2 * xy2)
    l2 = (p4x + p4y - 2 * C2_2) * inv_n2

    val = jnp.minimum(l1, l2) * jnp.float32(1.0 / _L)

    @pl.when(j == 0)
    def _():
        out_ref[...] = jnp.zeros_like(out_ref)

    out_ref[...] = out_ref[...] + val


@jax.jit
def kernel(xs, xt, P):
    pad = _NPAD - _N
    xs_r = jnp.pad(xs, ((0, pad), (0, 0))).T.reshape(3, _C, _R).swapaxes(1, 2)
    xt_r = jnp.pad(xt, ((0, pad), (0, 0))).T.reshape(3, _C, _R).swapaxes(1, 2)
    p_t = jnp.pad(P.T[:, None, :], ((0, 0), (0, 0), (0, _C - 3)))

    out = pl.pallas_call(
        _body,
        grid=(_L,),
        in_specs=[
            pl.BlockSpec((3, _R, _C), lambda j: (0, 0, 0)),
            pl.BlockSpec((3, _R, _C), lambda j: (0, 0, 0)),
            pl.BlockSpec((1, 1, _C), lambda j: (j, 0, 0)),
        ],
        out_specs=pl.BlockSpec((1, 1), lambda j: (0, 0)),
        out_shape=jax.ShapeDtypeStruct((1, 1), jnp.float32),
    )(xs_r, xt_r, p_t)
    return out[0, 0]
